# baseline (speedup 1.0000x reference)
import jax
import jax.numpy as jnp
from jax import lax
from jax.experimental import pallas as pl
from jax.experimental.pallas import tpu as pltpu
from jax.experimental.pallas import tpu_sc as plsc

_D, _H, _W = 16, 64, 64
_HP, _WP = 66, 67
_NB = 512
_BLK = _D * _H * _W
_PLANE = _H * _W
_HH = _H // 2
_HPIECE = _HH * _W
_HBLK = _D * _HPIECE
_OUT_BLK = _HP * _WP
_OUT_PAD = 4432
_GUARD = 16
_NW = 32
_BPW = _NB // _NW
_INV17 = float(1.0 / 17.0)


def _body(x_hbm, out_hbm, in0_v, in1_v, out_v, sem0, sem1):
    wid = lax.axis_index("s") * 2 + lax.axis_index("c")
    lane = lax.iota(jnp.int32, 16)
    even_lane = (lane & 1) == 0
    is0 = lane == 0
    le1 = lane <= 1
    is15 = lane == 15
    ge14 = lane >= 14
    zero = jnp.zeros((16,), jnp.float32)

    def zbody(i, c):
        out_v[pl.ds(i * 16, 16)] = zero
        return c

    lax.fori_loop(0, _OUT_PAD // 16, zbody, 0)

    def start_half(g, half, buf, sem):
        off = (wid * _BPW + g) * _BLK + half * _HPIECE
        for d in range(_D):
            pltpu.make_async_copy(
                x_hbm.at[pl.ds(off + d * _PLANE, _HPIECE)],
                buf.at[pl.ds(_GUARD + d * _HPIECE, _HPIECE)], sem).start()

    def wait_half(buf, sem):
        pltpu.make_async_copy(
            x_hbm.at[pl.ds(0, _HBLK)], buf.at[pl.ds(_GUARD, _HBLK)],
            sem).wait()

    def compute_half(buf, half):
        @plsc.parallel_loop(0, _HH)
        def hbody(h):
            hb = _GUARD + h * _W
            for w0 in (0, 16, 32, 48):
                acc = zero
                for d in range(_D):
                    base = hb + d * _HPIECE + w0
                    xv = buf[pl.ds(base, 16)]
                    l1 = buf[pl.ds(base + 1, 16)]
                    l2 = buf[pl.ds(base + 2, 16)]
                    r1 = buf[pl.ds(base - 1, 16)]
                    r2 = buf[pl.ds(base - 2, 16)]
                    if w0 == 48:
                        l1 = jnp.where(is15, 0.0, l1)
                        l2 = jnp.where(ge14, 0.0, l2)
                    if w0 == 0:
                        r1 = jnp.where(is0, 0.0, r1)
                        r2 = jnp.where(le1, 0.0, r2)
                    modd = (xv > r1) & (xv >= l1)
                    mev = (xv >= jnp.maximum(l1, l2)) | \
                        (xv > jnp.maximum(r1, r2))
                    xs = xv + xv / (1.0 + jnp.abs(xv))
                    t = jnp.where(even_lane, jnp.where(modd, xs, xv),
                                  jnp.where(mev, xs, xv))
                    acc = acc + t
                oa = (h + 1 + _HH * half) * _WP + 1 + w0
                out_v[pl.ds(oa, 16)] = acc * _INV17

    start_half(0, 0, in0_v, sem0)
    start_half(0, 1, in1_v, sem1)

    def gbody(g, c):
        bid = wid * _BPW + g
        wait_half(in0_v, sem0)
        compute_half(in0_v, 0)

        @pl.when(g + 1 < _BPW)
        def _():
            start_half(g + 1, 0, in0_v, sem0)

        wait_half(in1_v, sem1)
        compute_half(in1_v, 1)

        @pl.when(g + 1 < _BPW)
        def _():
            start_half(g + 1, 1, in1_v, sem1)

        pltpu.sync_copy(out_v, out_hbm.at[pl.ds(bid * _OUT_PAD, _OUT_PAD)])
        return c

    lax.fori_loop(0, _BPW, gbody, 0)


@jax.jit
def kernel(x):
    n, ch, d, h, w = x.shape
    xf = x.reshape(_NB * _BLK)
    run = pl.kernel(
        _body,
        out_type=jax.ShapeDtypeStruct((_NB * _OUT_PAD,), jnp.float32),
        mesh=plsc.VectorSubcoreMesh(core_axis_name="c", subcore_axis_name="s"),
        scratch_types=[
            pltpu.VMEM((_GUARD + _HBLK + _GUARD,), jnp.float32),
            pltpu.VMEM((_GUARD + _HBLK + _GUARD,), jnp.float32),
            pltpu.VMEM((_OUT_PAD,), jnp.float32),
            pltpu.SemaphoreType.DMA,
            pltpu.SemaphoreType.DMA,
        ],
    )
    out = run(xf)
    return out.reshape(_NB, _OUT_PAD)[:, :_OUT_BLK].reshape(n, ch, _HP, _WP)


# speedup vs baseline: 1.0004x; 1.0004x over previous
import jax
import jax.numpy as jnp
from jax import lax
from jax.experimental import pallas as pl
from jax.experimental.pallas import tpu as pltpu
from jax.experimental.pallas import tpu_sc as plsc

_D, _H, _W = 16, 64, 64
_HP, _WP = 66, 67
_NB = 512
_BLK = _D * _H * _W
_PLANE = _H * _W
_HH = _H // 2
_HPIECE = _HH * _W
_HBLK = _D * _HPIECE
_OUT_BLK = _HP * _WP
_OUT_PAD = 4432
_GUARD = 16
_NW = 32
_BPW = _NB // _NW
_INV17 = float(1.0 / 17.0)


def _body(x_hbm, out_hbm, in0_v, in1_v, out_v, sem0, sem1):
    wid = lax.axis_index("s") * 2 + lax.axis_index("c")
    lane = lax.iota(jnp.int32, 16)
    even_lane = (lane & 1) == 0
    is0 = lane == 0
    le1 = lane <= 1
    is15 = lane == 15
    ge14 = lane >= 14
    zero = jnp.zeros((16,), jnp.float32)

    def zbody(i, c):
        out_v[pl.ds(i * 16, 16)] = zero
        return c

    lax.fori_loop(0, _OUT_PAD // 16, zbody, 0)

    def start_half(g, half, buf, sem):
        off = (wid * _BPW + g) * _BLK + half * _HPIECE
        for d in range(_D):
            pltpu.make_async_copy(
                x_hbm.at[pl.ds(off + d * _PLANE, _HPIECE)],
                buf.at[pl.ds(_GUARD + d * _HPIECE, _HPIECE)], sem).start()

    def wait_half(buf, sem):
        pltpu.make_async_copy(
            x_hbm.at[pl.ds(0, _HBLK)], buf.at[pl.ds(_GUARD, _HBLK)],
            sem).wait()

    def compute_half(buf, half):
        def hbody(h, cc):
            hb = _GUARD + h * _W
            for w0 in (0, 16, 32, 48):
                acc = zero
                for d in range(_D):
                    base = hb + d * _HPIECE + w0
                    xv = buf[pl.ds(base, 16)]
                    l1 = buf[pl.ds(base + 1, 16)]
                    l2 = buf[pl.ds(base + 2, 16)]
                    r1 = buf[pl.ds(base - 1, 16)]
                    r2 = buf[pl.ds(base - 2, 16)]
                    if w0 == 48:
                        l1 = jnp.where(is15, 0.0, l1)
                        l2 = jnp.where(ge14, 0.0, l2)
                    if w0 == 0:
                        r1 = jnp.where(is0, 0.0, r1)
                        r2 = jnp.where(le1, 0.0, r2)
                    modd = (xv > r1) & (xv >= l1)
                    mev = (xv >= jnp.maximum(l1, l2)) | \
                        (xv > jnp.maximum(r1, r2))
                    xs = xv + xv / (1.0 + jnp.abs(xv))
                    t = jnp.where(even_lane, jnp.where(modd, xs, xv),
                                  jnp.where(mev, xs, xv))
                    acc = acc + t
                oa = (h + 1 + _HH * half) * _WP + 1 + w0
                out_v[pl.ds(oa, 16)] = acc * _INV17
            return cc

        lax.fori_loop(0, _HH, hbody, 0)

    start_half(0, 0, in0_v, sem0)
    start_half(0, 1, in1_v, sem1)

    def gbody(g, c):
        bid = wid * _BPW + g
        wait_half(in0_v, sem0)
        compute_half(in0_v, 0)

        @pl.when(g + 1 < _BPW)
        def _():
            start_half(g + 1, 0, in0_v, sem0)

        wait_half(in1_v, sem1)
        compute_half(in1_v, 1)

        @pl.when(g + 1 < _BPW)
        def _():
            start_half(g + 1, 1, in1_v, sem1)

        pltpu.sync_copy(out_v, out_hbm.at[pl.ds(bid * _OUT_PAD, _OUT_PAD)])
        return c

    lax.fori_loop(0, _BPW, gbody, 0)


@jax.jit
def kernel(x):
    n, ch, d, h, w = x.shape
    xf = x.reshape(_NB * _BLK)
    run = pl.kernel(
        _body,
        out_type=jax.ShapeDtypeStruct((_NB * _OUT_PAD,), jnp.float32),
        mesh=plsc.VectorSubcoreMesh(core_axis_name="c", subcore_axis_name="s"),
        scratch_types=[
            pltpu.VMEM((_GUARD + _HBLK + _GUARD,), jnp.float32),
            pltpu.VMEM((_GUARD + _HBLK + _GUARD,), jnp.float32),
            pltpu.VMEM((_OUT_PAD,), jnp.float32),
            pltpu.SemaphoreType.DMA,
            pltpu.SemaphoreType.DMA,
        ],
    )
    out = run(xf)
    return out.reshape(_NB, _OUT_PAD)[:, :_OUT_BLK].reshape(n, ch, _HP, _WP)
